# baseline (speedup 1.0000x reference)
import functools

import jax
import jax.numpy as jnp
from jax import lax
from jax.experimental import pallas as pl
from jax.experimental.pallas import tpu as pltpu
from jax.experimental.pallas import tpu_sc as plsc

DEPTH = 1000
B0, B1 = 1024, 26
NC, NS = 2, 16
NW = NC * NS
ROWS = B1 * DEPTH
CH = 40
NCH = ROWS // CH
NT = -(-NCH // NW)
LAST_W = NCH - (NT - 1) * NW


def _one_hot_sc(xg, zero_chunk):
    mesh = plsc.VectorSubcoreMesh(core_axis_name="c", subcore_axis_name="s")

    @functools.partial(
        pl.kernel,
        mesh=mesh,
        out_type=jax.ShapeDtypeStruct((ROWS, B0), jnp.int32),
        compiler_params=pltpu.CompilerParams(needs_layout_passes=False),
        scratch_types=[
            pltpu.VMEM((NT * 2 * B0,), jnp.int32),
            pltpu.VMEM((CH, B0), jnp.int32),
            pltpu.VMEM((CH, B0), jnp.int32),
            pltpu.SemaphoreType.DMA,
            pltpu.SemaphoreType.DMA,
            pltpu.SemaphoreType.DMA,
        ],
    )
    def k(xg_hbm, z_hbm, out_hbm, xv, buf0, buf1, sem0, sem1, sem_x):
        wid = lax.axis_index("s") * NC + lax.axis_index("c")
        bufs = (buf0, buf1)
        sems = (sem0, sem1)

        def chunk_row0(t):
            return (t * NW + wid) * CH

        for t in range(NT):
            r0 = chunk_row0(t)
            j0 = jnp.minimum(r0 // DEPTH, B1 - 1)
            for jj in range(2):
                jle = jnp.minimum(j0 + jj, B1 - 1)
                pltpu.async_copy(xg_hbm.at[pl.ds(jle * B0, B0)],
                                 xv.at[pl.ds((2 * t + jj) * B0, B0)], sem_x)
        for p in range(2):
            pltpu.async_copy(z_hbm, bufs[p], sems[p])
        for _ in range(NT * 2):
            pltpu.make_async_copy(xg_hbm.at[pl.ds(0, B0)],
                                  xv.at[pl.ds(0, B0)], sem_x).wait()

        lane = lax.iota(jnp.int32, 16)
        ones = jnp.full((16,), 1, jnp.int32)
        zeros = jnp.zeros((16,), jnp.int32)

        def scatter_chunk(p, t, r0, vals):
            for jj in range(2):
                for s in range(B0 // 16):
                    rows = xv[pl.ds((2 * t + jj) * B0 + s * 16, 16)]
                    mask = (rows >= r0) & (rows < r0 + CH)
                    plsc.store_scatter(bufs[p], [rows - r0, lane + s * 16],
                                       vals, mask=mask)

        def dst(r0):
            return out_hbm.at[pl.ds(r0, CH)]

        def drain(p):
            pltpu.make_async_copy(bufs[p], out_hbm.at[pl.ds(0, CH)],
                                  sems[p]).wait()

        def round_body(t, t_static_pair):
            for p in range(2):
                tt = t + p
                r0 = chunk_row0(tt)
                if t_static_pair == 0 and p == 0:
                    drain(0)
                    scatter_chunk(0, tt, r0, ones)
                    pltpu.async_copy(bufs[0], dst(r0), sems[0])
                elif t_static_pair == 0 and p == 1:
                    drain(1)
                    scatter_chunk(1, tt, r0, ones)
                    drain(0)
                    pltpu.async_copy(bufs[1], dst(r0), sems[1])
                else:
                    scatter_chunk(p, tt - 2, chunk_row0(tt - 2), zeros)
                    scatter_chunk(p, tt, r0, ones)
                    drain(1 - p)
                    pltpu.async_copy(bufs[p], dst(r0), sems[p])

        round_body(0, 0)

        def pair(q, carry):
            round_body(2 * q, 1)
            return carry

        lax.fori_loop(1, NT // 2, pair, 0)

        @pl.when(wid < LAST_W)
        def _():
            t = NT - 1
            r0 = chunk_row0(t)
            scatter_chunk(0, t - 2, chunk_row0(t - 2), zeros)
            scatter_chunk(0, t, r0, ones)
            drain(1)
            pltpu.async_copy(bufs[0], dst(r0), sems[0])

        @pl.when(wid < LAST_W)
        def _():
            drain(0)

        @pl.when(wid >= LAST_W)
        def _():
            drain(1)

    return k(xg, zero_chunk)


def kernel(x):
    xg = (x + DEPTH * jnp.arange(B1, dtype=jnp.int32)[None, :]).T.reshape(-1)
    zero_chunk = jnp.zeros((CH, B0), jnp.int32)
    out_flat = _one_hot_sc(xg, zero_chunk)
    return jnp.transpose(out_flat.reshape(B1, DEPTH, B0), (2, 0, 1))


# speedup vs baseline: 1.7622x; 1.7622x over previous
import functools

import jax
import jax.numpy as jnp
from jax import lax
from jax.experimental import pallas as pl
from jax.experimental.pallas import tpu as pltpu
from jax.experimental.pallas import tpu_sc as plsc

DEPTH = 1000
B0, B1 = 1024, 26
NC, NS = 2, 16
NW = NC * NS
COLS = 128
NCHUNK = B1 * (B0 // COLS)
NT = -(-NCHUNK // NW)


def _one_hot_sc(xt_flat, zero_chunk):
    mesh = plsc.VectorSubcoreMesh(core_axis_name="c", subcore_axis_name="s")

    @functools.partial(
        pl.kernel,
        mesh=mesh,
        out_type=jax.ShapeDtypeStruct((B1, DEPTH, B0), jnp.int32),
        compiler_params=pltpu.CompilerParams(needs_layout_passes=False),
        scratch_types=[
            pltpu.VMEM((NT * COLS,), jnp.int32),
            pltpu.VMEM((DEPTH, COLS), jnp.int32),
            pltpu.SemaphoreType.DMA,
            pltpu.SemaphoreType.DMA,
        ],
    )
    def k(xt_hbm, z_hbm, out_hbm, xv, buf, sem, sem_x):
        wid = lax.axis_index("s") * NC + lax.axis_index("c")
        g_last = (NT - 1) * NW + (wid & (NW // 2 - 1))

        def chunk_id(t):
            return g_last if t == NT - 1 else t * NW + wid

        for t in range(NT):
            pltpu.async_copy(xt_hbm.at[pl.ds(chunk_id(t) * COLS, COLS)],
                             xv.at[pl.ds(t * COLS, COLS)], sem_x)

        pltpu.async_copy(z_hbm, buf, sem)

        for t in range(NT):
            pltpu.make_async_copy(xt_hbm.at[pl.ds(0, COLS)],
                                  xv.at[pl.ds(0, COLS)], sem_x).wait()

        lane = lax.iota(jnp.int32, 16)
        ones = jnp.full((16,), 1, jnp.int32)
        zeros = jnp.zeros((16,), jnp.int32)
        upper = wid >= NW // 2

        def scatter_chunk(t, vals, mask=None):
            for s in range(COLS // 16):
                rows = xv[pl.ds(t * COLS + s * 16, 16)]
                m = None if mask is None else mask(rows)
                plsc.store_scatter(buf, [rows, lane + s * 16], vals, mask=m)

        for t in range(NT):
            g = chunk_id(t)
            j = g // (B0 // COLS)
            c = g % (B0 // COLS)

            pltpu.make_async_copy(
                buf, out_hbm.at[0, :, pl.ds(0, COLS)], sem).wait()
            if t > 0:
                scatter_chunk(t - 1, zeros)
            if t < NT - 1:
                scatter_chunk(t, ones)
                pltpu.async_copy(
                    buf, out_hbm.at[j, :, pl.ds(c * COLS, COLS)], sem)
            else:
                for r0, nr, cond in ((0, 496, ~upper), (496, 504, upper)):
                    @pl.when(cond)
                    def _(r0=r0, nr=nr):
                        scatter_chunk(
                            t, ones,
                            mask=lambda rows: ((rows >= r0) & (rows < r0 + nr)))
                        pltpu.async_copy(
                            buf.at[pl.ds(r0, nr)],
                            out_hbm.at[j, pl.ds(r0, nr),
                                       pl.ds(c * COLS, COLS)],
                            sem)

        for r0, nr, cond in ((0, 496, ~upper), (496, 504, upper)):
            @pl.when(cond)
            def _(r0=r0, nr=nr):
                pltpu.make_async_copy(
                    buf.at[pl.ds(r0, nr)],
                    out_hbm.at[0, pl.ds(r0, nr), pl.ds(0, COLS)], sem).wait()

    return k(xt_flat, zero_chunk)


def kernel(x):
    xt_flat = x.T.reshape(-1)
    zero_chunk = jnp.zeros((DEPTH, COLS), jnp.int32)
    out_t = _one_hot_sc(xt_flat, zero_chunk)
    return jnp.transpose(out_t, (2, 0, 1))
